# baseline (speedup 1.0000x reference)
import functools

import jax
import jax.numpy as jnp
from jax import lax
from jax.experimental import pallas as pl
from jax.experimental.pallas import tpu as pltpu
from jax.experimental.pallas import tpu_sc as plsc

D3 = 256
D_MODEL = 768
SEQ = 81
REP = 32
BR = SEQ * REP


SEQ_PAD = 96


@functools.partial(
    pl.kernel,
    mesh=plsc.VectorSubcoreMesh(core_axis_name="c", subcore_axis_name="s"),
    out_type=[jax.ShapeDtypeStruct((SEQ_PAD, D3), jnp.float32)] * 3,
    scratch_types=[
        pltpu.VMEM((SEQ_PAD,), jnp.int32),
        pltpu.VMEM((SEQ_PAD, D3), jnp.float32),
        pltpu.SemaphoreType.DMA,
    ],
)
def _pe_gather(rows_hbm, cols_hbm, boxes_hbm,
               row_tab_hbm, col_tab_hbm, box_tab_hbm,
               row_pe_hbm, col_pe_hbm, box_pe_hbm,
               idx_v, pe_v, sem):
    wid = lax.axis_index("s") * 2 + lax.axis_index("c")

    def gather_one(idx_hbm, tab_hbm, out_hbm):
        pltpu.sync_copy(idx_hbm, idx_v)
        pltpu.async_copy(tab_hbm.at[idx_v], pe_v, sem).wait()
        pltpu.sync_copy(pe_v, out_hbm)

    @pl.when(wid == 0)
    def _():
        gather_one(rows_hbm, row_tab_hbm, row_pe_hbm)

    @pl.when(wid == 1)
    def _():
        gather_one(cols_hbm, col_tab_hbm, col_pe_hbm)

    @pl.when(wid == 2)
    def _():
        gather_one(boxes_hbm, box_tab_hbm, box_pe_hbm)


def _add_kernel(rpe_ref, cpe_ref, bpe_ref, x_ref, out_ref, ptile_ref):
    @pl.when(pl.program_id(0) == 0)
    def _build_ptile():
        for k in range(REP):
            base = SEQ * k
            ptile_ref[pl.ds(base, SEQ), 0:D3] = rpe_ref[...]
            ptile_ref[pl.ds(base, SEQ), D3:2 * D3] = cpe_ref[...]
            ptile_ref[pl.ds(base, SEQ), 2 * D3:D_MODEL] = bpe_ref[...]

    out_ref[...] = x_ref[...] + ptile_ref[...]


@jax.jit
def kernel(x, row_table, col_table, box_table, rows, cols, boxes):
    pad = jnp.zeros((SEQ_PAD - SEQ,), jnp.int32)
    row_pe, col_pe, box_pe = _pe_gather(
        jnp.concatenate([rows, pad]), jnp.concatenate([cols, pad]),
        jnp.concatenate([boxes, pad]), row_table, col_table, box_table)
    row_pe, col_pe, box_pe = row_pe[:SEQ], col_pe[:SEQ], box_pe[:SEQ]
    b = x.shape[0]
    x2 = x.reshape(b * SEQ, D_MODEL)
    pe_spec = pl.BlockSpec((SEQ, D3), lambda i: (0, 0))
    out2 = pl.pallas_call(
        _add_kernel,
        grid=(b * SEQ // BR,),
        in_specs=[
            pe_spec, pe_spec, pe_spec,
            pl.BlockSpec((BR, D_MODEL), lambda i: (i, 0)),
        ],
        out_specs=pl.BlockSpec((BR, D_MODEL), lambda i: (i, 0)),
        out_shape=jax.ShapeDtypeStruct(x2.shape, x2.dtype),
        scratch_shapes=[pltpu.VMEM((BR, D_MODEL), jnp.float32)],
        compiler_params=pltpu.CompilerParams(
            dimension_semantics=("arbitrary",),
        ),
    )(row_pe, col_pe, box_pe, x2)
    return out2.reshape(x.shape)


# speedup vs baseline: 1.6497x; 1.6497x over previous
import functools

import jax
import jax.numpy as jnp
from jax import lax
from jax.experimental import pallas as pl
from jax.experimental.pallas import tpu as pltpu
from jax.experimental.pallas import tpu_sc as plsc

D3 = 256
D_MODEL = 768
SEQ = 81
REP = 32
BR = SEQ * REP


SEQ_PAD = 96


@functools.partial(
    pl.kernel,
    mesh=plsc.VectorSubcoreMesh(core_axis_name="c", subcore_axis_name="s"),
    out_type=[jax.ShapeDtypeStruct((SEQ_PAD, D3), jnp.float32)] * 3,
    scratch_types=[
        pltpu.VMEM((SEQ_PAD,), jnp.int32),
        pltpu.VMEM((SEQ_PAD, D3), jnp.float32),
        pltpu.SemaphoreType.DMA,
    ],
)
def _pe_gather(rows_hbm, cols_hbm, boxes_hbm,
               row_tab_hbm, col_tab_hbm, box_tab_hbm,
               row_pe_hbm, col_pe_hbm, box_pe_hbm,
               idx_v, pe_v, sem):
    wid = lax.axis_index("s") * 2 + lax.axis_index("c")

    def gather_one(idx_hbm, tab_hbm, out_hbm):
        pltpu.sync_copy(idx_hbm, idx_v)
        pltpu.async_copy(tab_hbm.at[idx_v], pe_v, sem).wait()
        pltpu.sync_copy(pe_v, out_hbm)

    @pl.when(wid == 0)
    def _():
        gather_one(rows_hbm, row_tab_hbm, row_pe_hbm)

    @pl.when(wid == 1)
    def _():
        gather_one(cols_hbm, col_tab_hbm, col_pe_hbm)

    @pl.when(wid == 2)
    def _():
        gather_one(boxes_hbm, box_tab_hbm, box_pe_hbm)


def _add_kernel(rpe_ref, cpe_ref, bpe_ref, x_ref, out_ref, pe_ref):
    @pl.when(pl.program_id(0) == 0)
    def _build_pe():
        pe_ref[:, 0:D3] = rpe_ref[...]
        pe_ref[:, D3:2 * D3] = cpe_ref[...]
        pe_ref[:, 2 * D3:D_MODEL] = bpe_ref[...]

    out_ref[...] = x_ref[...] + pe_ref[...][None, :, :]


@jax.jit
def kernel(x, row_table, col_table, box_table, rows, cols, boxes):
    pad = jnp.zeros((SEQ_PAD - SEQ,), jnp.int32)
    row_pe, col_pe, box_pe = _pe_gather(
        jnp.concatenate([rows, pad]), jnp.concatenate([cols, pad]),
        jnp.concatenate([boxes, pad]), row_table, col_table, box_table)
    row_pe, col_pe, box_pe = row_pe[:SEQ], col_pe[:SEQ], box_pe[:SEQ]
    b = x.shape[0]
    pe_spec = pl.BlockSpec((SEQ, D3), lambda i: (0, 0))
    return pl.pallas_call(
        _add_kernel,
        grid=(b // REP,),
        in_specs=[
            pe_spec, pe_spec, pe_spec,
            pl.BlockSpec((REP, SEQ, D_MODEL), lambda i: (i, 0, 0)),
        ],
        out_specs=pl.BlockSpec((REP, SEQ, D_MODEL), lambda i: (i, 0, 0)),
        out_shape=jax.ShapeDtypeStruct(x.shape, x.dtype),
        scratch_shapes=[pltpu.VMEM((SEQ, D_MODEL), jnp.float32)],
        compiler_params=pltpu.CompilerParams(
            dimension_semantics=("arbitrary",),
        ),
    )(row_pe, col_pe, box_pe, x)


# speedup vs baseline: 1.6634x; 1.0083x over previous
import jax
import jax.numpy as jnp
from jax.experimental import pallas as pl
from jax.experimental.pallas import tpu as pltpu

D3 = 256
D_MODEL = 768
SEQ = 81
REP = 32


def _pe_add_kernel(rows_ref, cols_ref, boxes_ref,
                   row_tab_ref, col_tab_ref, box_tab_ref,
                   x_ref, out_ref, pe_ref):
    @pl.when(pl.program_id(0) == 0)
    def _build_pe():
        iota = jax.lax.broadcasted_iota(jnp.int32, (SEQ, 9), 1)
        oh_rows = (rows_ref[...] == iota).astype(jnp.float32)
        oh_cols = (cols_ref[...] == iota).astype(jnp.float32)
        oh_boxes = (boxes_ref[...] == iota).astype(jnp.float32)
        pe_ref[:, 0:D3] = jnp.dot(oh_rows, row_tab_ref[...],
                                  preferred_element_type=jnp.float32)
        pe_ref[:, D3:2 * D3] = jnp.dot(oh_cols, col_tab_ref[...],
                                       preferred_element_type=jnp.float32)
        pe_ref[:, 2 * D3:D_MODEL] = jnp.dot(oh_boxes, box_tab_ref[...],
                                            preferred_element_type=jnp.float32)

    out_ref[...] = x_ref[...] + pe_ref[...][None, :, :]


@jax.jit
def kernel(x, row_table, col_table, box_table, rows, cols, boxes):
    b = x.shape[0]
    full = lambda shape: pl.BlockSpec(shape, lambda i: (0,) * len(shape))
    return pl.pallas_call(
        _pe_add_kernel,
        grid=(b // REP,),
        in_specs=[
            full((SEQ, 1)),
            full((SEQ, 1)),
            full((SEQ, 1)),
            full((9, D3)),
            full((9, D3)),
            full((9, D_MODEL - 2 * D3)),
            pl.BlockSpec((REP, SEQ, D_MODEL), lambda i: (i, 0, 0)),
        ],
        out_specs=pl.BlockSpec((REP, SEQ, D_MODEL), lambda i: (i, 0, 0)),
        out_shape=jax.ShapeDtypeStruct(x.shape, x.dtype),
        scratch_shapes=[pltpu.VMEM((SEQ, D_MODEL), jnp.float32)],
        compiler_params=pltpu.CompilerParams(
            dimension_semantics=("arbitrary",),
        ),
    )(rows.reshape(SEQ, 1), cols.reshape(SEQ, 1), boxes.reshape(SEQ, 1),
      row_table, col_table, box_table, x)
